# baseline (speedup 1.0000x reference)
import functools

import jax
import jax.numpy as jnp
from jax import lax
from jax.experimental import pallas as pl
from jax.experimental.pallas import tpu as pltpu
from jax.experimental.pallas import tpu_sc as plsc

NUM_EMB = 8192
D_EMB = 256
B = 20000
QK = NUM_EMB // 4

NW = 32
CH = 128
NFULL = B // CH
TAIL = B - NFULL * CH
KMAX = 5

BP = 20480
TR = 1024
NT = BP // TR


def _vq_body(lohi_ref, g_ref, rep_ref, w_ref, wsq_ref,
             enc_ref, loss_ref):
    t = pl.program_id(0)
    lo = lohi_ref[0, t]
    hi = lohi_ref[1, t]
    rep = rep_ref[...]
    gv = g_ref[0, 0, :]
    rsq = jnp.sum(rep * rep, axis=1)
    ids = lax.broadcasted_iota(jnp.int32, (TR, QK), 1)

    def qstep(q, carry):
        best, besti = carry
        wq = w_ref[q]
        y2 = lax.dot_general(rep, wq, (((1,), (1,)), ((), ())),
                             preferred_element_type=jnp.float32)
        wsq = wsq_ref[q]
        d = (rsq[:, None] + wsq) - y2
        minv = jnp.min(d, axis=1)
        mini = jnp.min(jnp.where(d <= minv[:, None], ids, QK), axis=1)
        msk = gv == q
        best = jnp.where(msk, minv, best)
        besti = jnp.where(msk, mini + q * QK, besti)
        return best, besti

    best0 = jnp.full((TR,), jnp.inf, jnp.float32)
    besti0 = jnp.zeros((TR,), jnp.int32)
    best, besti = lax.fori_loop(lo, hi + 1, qstep, (best0, besti0))
    enc_ref[0, 0, :] = besti
    part = jnp.sum(jnp.where(gv >= 0, best, 0.0))

    @pl.when(t == 0)
    def _():
        loss_ref[0, 0] = 0.0

    loss_ref[0, 0] += part


def _encode(lohi, gs3, rep_s, w4, wsq43):
    enc3, loss = pl.pallas_call(
        _vq_body,
        grid_spec=pltpu.PrefetchScalarGridSpec(
            num_scalar_prefetch=1,
            grid=(NT,),
            in_specs=[
                pl.BlockSpec((1, 1, TR), lambda t, s: (t, 0, 0)),
                pl.BlockSpec((TR, D_EMB), lambda t, s: (t, 0)),
                pl.BlockSpec((4, QK, D_EMB), lambda t, s: (0, 0, 0)),
                pl.BlockSpec((4, 1, QK), lambda t, s: (0, 0, 0)),
            ],
            out_specs=[
                pl.BlockSpec((1, 1, TR), lambda t, s: (t, 0, 0)),
                pl.BlockSpec(memory_space=pltpu.SMEM),
            ],
        ),
        out_shape=[
            jax.ShapeDtypeStruct((NT, 1, TR), jnp.int32),
            jax.ShapeDtypeStruct((1, 1), jnp.float32),
        ],
    )(lohi, gs3, rep_s, w4, wsq43)
    return enc3.reshape(BP), loss[0, 0]


def _mesh():
    return plsc.VectorSubcoreMesh(core_axis_name="c", subcore_axis_name="s")


_RING_SCRATCH = (
    [pltpu.VMEM((CH,), jnp.int32) for _ in range(KMAX)]
    + [pltpu.VMEM((TAIL,), jnp.int32)]
    + [pltpu.VMEM((CH, D_EMB), jnp.float32) for _ in range(3)]
    + [pltpu.VMEM((TAIL, D_EMB), jnp.float32)]
    + [pltpu.SemaphoreType.DMA] * 9
)

_NLAST = NFULL - (KMAX - 1) * NW


def _ring_pipeline(wid, idx_hbm, idx_refs, it, bufs, bt, isem, rsems, ssems,
                   lit, st, fire_move, fire_out, tail_fn):
    nfix = KMAX - 1
    ixd = [None] * nfix
    for k in range(nfix):
        off = (wid + NW * k) * CH
        ixd[k] = pltpu.async_copy(idx_hbm.at[pl.ds(off, CH)], idx_refs[k],
                                  isem)
    for k in range(nfix):
        ixd[k].wait()

    md, sd = [None] * nfix, [None] * nfix
    for k in range(3):
        md[k] = fire_move(k, k % 3)
    for k in range(nfix):
        md[k].wait()
        sd[k] = fire_out(k, k % 3)
        if k + 3 < nfix:
            sd[k].wait()
            md[k + 3] = fire_move(k + 3, (k + 3) % 3)
    for k in range(max(0, nfix - 3), nfix):
        sd[k].wait()

    @pl.when(wid < _NLAST)
    def _():
        off = (wid + NW * nfix) * CH
        pltpu.sync_copy(idx_hbm.at[pl.ds(off, CH)], idx_refs[nfix])
        md4 = fire_move(nfix, 0)
        md4.wait()
        fire_out(nfix, 0).wait()

    @pl.when(wid == NW - 1)
    def _():
        pltpu.sync_copy(idx_hbm.at[pl.ds(NFULL * CH, TAIL)], it)
        tail_fn()


@functools.lru_cache(maxsize=1)
def _sc_sort_scatter_fn():
    @functools.partial(
        pl.kernel,
        mesh=_mesh(),
        out_type=jax.ShapeDtypeStruct((BP, D_EMB), jnp.float32),
        scratch_types=list(_RING_SCRATCH),
    )
    def _go(rep_hbm, pos_hbm, out_hbm, i0, i1, i2, i3, i4, it, b0, b1, b2,
            bt, isem, r0, r1, r2, s0, s1, s2, lit, st):
        wid = lax.axis_index("s") * 2 + lax.axis_index("c")
        idx_refs, bufs = (i0, i1, i2, i3, i4), (b0, b1, b2)
        rsems, ssems = (r0, r1, r2), (s0, s1, s2)

        def fire_move(k, r):
            off = (wid + NW * k) * CH
            return pltpu.async_copy(rep_hbm.at[pl.ds(off, CH), :], bufs[r],
                                    rsems[r])

        def fire_out(k, r):
            return pltpu.async_copy(bufs[r], out_hbm.at[idx_refs[k]],
                                    ssems[r])

        def tail_fn():
            off = NFULL * CH
            pltpu.async_copy(rep_hbm.at[pl.ds(off, TAIL), :], bt, lit).wait()
            pltpu.async_copy(bt, out_hbm.at[it], st).wait()

        _ring_pipeline(wid, pos_hbm, idx_refs, it, bufs, bt, isem, rsems,
                       ssems, lit, st, fire_move, fire_out, tail_fn)

    return _go


@functools.lru_cache(maxsize=1)
def _sc_lookup_fn():
    @functools.partial(
        pl.kernel,
        mesh=_mesh(),
        out_type=jax.ShapeDtypeStruct((B, D_EMB), jnp.float32),
        scratch_types=list(_RING_SCRATCH),
    )
    def _go(w_hbm, enc_hbm, out_hbm, i0, i1, i2, i3, i4, it, b0, b1, b2,
            bt, isem, r0, r1, r2, s0, s1, s2, lit, st):
        wid = lax.axis_index("s") * 2 + lax.axis_index("c")
        idx_refs, bufs = (i0, i1, i2, i3, i4), (b0, b1, b2)
        rsems, ssems = (r0, r1, r2), (s0, s1, s2)

        def fire_move(k, r):
            return pltpu.async_copy(w_hbm.at[idx_refs[k]], bufs[r], rsems[r])

        def fire_out(k, r):
            off = (wid + NW * k) * CH
            return pltpu.async_copy(bufs[r], out_hbm.at[pl.ds(off, CH), :],
                                    ssems[r])

        def tail_fn():
            off = NFULL * CH
            pltpu.async_copy(w_hbm.at[it], bt, lit).wait()
            pltpu.async_copy(bt, out_hbm.at[pl.ds(off, TAIL), :], st).wait()

        _ring_pipeline(wid, enc_hbm, idx_refs, it, bufs, bt, isem, rsems,
                       ssems, lit, st, fire_move, fire_out, tail_fn)

    return _go


def kernel(node_type, node_representation, W):
    rep = node_representation.astype(jnp.float32)
    w = W.astype(jnp.float32)
    nt = node_type.astype(jnp.int32)
    g = jnp.where(nt == 5, 0, jnp.where(nt == 6, 1, jnp.where(nt == 7, 2, 3)))

    onehot = (g[:, None] == jnp.arange(4, dtype=jnp.int32)[None, :])
    cnt = onehot.astype(jnp.int32)
    totals = cnt[B - 1]
    offsets = jnp.concatenate(
        [jnp.zeros((1,), jnp.int32), jnp.cumsum(totals)[:3]])
    rank = jnp.take_along_axis(cnt, g[:, None], axis=1)[:, 0] - 1
    pos = offsets[g] + rank
    co = jnp.cumsum(totals)[:3]
    gs = jnp.sum(
        (jnp.arange(B, dtype=jnp.int32)[:, None] >= co[None, :]),
        axis=1).astype(jnp.int32)

    wsq = jnp.sum(w ** 2, axis=1)

    rep_s = _sc_sort_scatter_fn()(rep, pos)

    pad_g = jnp.full((BP - B,), -1, jnp.int32)
    gs3 = jnp.concatenate([gs, pad_g]).reshape(NT, 1, TR)
    tstart = jnp.arange(NT, dtype=jnp.int32) * TR
    lo = gs[jnp.minimum(tstart, B - 1)]
    hi = gs[jnp.minimum(tstart + TR - 1, B - 1)]
    lohi = jnp.stack([lo, hi])

    w4 = (w + w).reshape(4, QK, D_EMB)
    wsq43 = wsq.reshape(4, 1, QK)
    enc_s, loss_sum = _encode(lohi, gs3, rep_s, w4, wsq43)
    enc = enc_s[pos]

    quantized = _sc_lookup_fn()(w, enc)
    vq_loss = loss_sum / jnp.float32(B * D_EMB)
    commitment_loss = jnp.float32(0.25) * vq_loss
    return (vq_loss, commitment_loss, enc, quantized)


# speedup vs baseline: 9.3546x; 9.3546x over previous
import functools

import jax
import jax.numpy as jnp
from jax import lax
from jax.experimental import pallas as pl
from jax.experimental.pallas import tpu as pltpu
from jax.experimental.pallas import tpu_sc as plsc

NUM_EMB = 8192
D_EMB = 256
B = 20000
QK = NUM_EMB // 4

NW = 32
CH = 128
NFULL = B // CH
TAIL = B - NFULL * CH
KMAX = 5

BP = 20480
TR = 1024
NT = BP // TR


def _vq_body(lohi_ref, g_ref, rep_ref, w_ref, wsq_ref,
             enc_ref, loss_ref):
    t = pl.program_id(0)
    lo = lohi_ref[0, t]
    hi = lohi_ref[1, t]
    rep = rep_ref[...]
    gv = g_ref[0, 0, :]
    rsq = jnp.sum(rep * rep, axis=1)
    ids = lax.broadcasted_iota(jnp.int32, (TR, QK), 1)

    def qstep(q, carry):
        best, besti = carry
        wq = w_ref[q]
        y2 = lax.dot_general(rep, wq, (((1,), (1,)), ((), ())),
                             preferred_element_type=jnp.float32)
        wsq = wsq_ref[q]
        d = (rsq[:, None] + wsq) - y2
        minv = jnp.min(d, axis=1)
        mini = jnp.min(jnp.where(d <= minv[:, None], ids, QK), axis=1)
        msk = gv == q
        best = jnp.where(msk, minv, best)
        besti = jnp.where(msk, mini + q * QK, besti)
        return best, besti

    best0 = jnp.full((TR,), jnp.inf, jnp.float32)
    besti0 = jnp.zeros((TR,), jnp.int32)
    best, besti = lax.fori_loop(lo, hi + 1, qstep, (best0, besti0))
    enc_ref[0, 0, :] = besti
    part = jnp.sum(jnp.where(gv >= 0, best, 0.0))

    @pl.when(t == 0)
    def _():
        loss_ref[0, 0] = 0.0

    loss_ref[0, 0] += part


def _encode(lohi, gs3, rep_s, w4, wsq43):
    enc3, loss = pl.pallas_call(
        _vq_body,
        grid_spec=pltpu.PrefetchScalarGridSpec(
            num_scalar_prefetch=1,
            grid=(NT,),
            in_specs=[
                pl.BlockSpec((1, 1, TR), lambda t, s: (t, 0, 0)),
                pl.BlockSpec((TR, D_EMB), lambda t, s: (t, 0)),
                pl.BlockSpec((4, QK, D_EMB), lambda t, s: (0, 0, 0)),
                pl.BlockSpec((4, 1, QK), lambda t, s: (0, 0, 0)),
            ],
            out_specs=[
                pl.BlockSpec((1, 1, TR), lambda t, s: (t, 0, 0)),
                pl.BlockSpec(memory_space=pltpu.SMEM),
            ],
        ),
        out_shape=[
            jax.ShapeDtypeStruct((NT, 1, TR), jnp.int32),
            jax.ShapeDtypeStruct((1, 1), jnp.float32),
        ],
    )(lohi, gs3, rep_s, w4, wsq43)
    return enc3.reshape(BP), loss[0, 0]


def _mesh():
    return plsc.VectorSubcoreMesh(core_axis_name="c", subcore_axis_name="s")


_RING_SCRATCH = (
    [pltpu.VMEM((CH,), jnp.int32) for _ in range(KMAX)]
    + [pltpu.VMEM((TAIL,), jnp.int32)]
    + [pltpu.VMEM((CH, D_EMB), jnp.float32) for _ in range(3)]
    + [pltpu.VMEM((TAIL, D_EMB), jnp.float32)]
    + [pltpu.SemaphoreType.DMA] * 9
)

_NLAST = NFULL - (KMAX - 1) * NW


def _ring_pipeline(wid, idx_hbm, idx_refs, it, bufs, bt, isem, rsems, ssems,
                   lit, st, fire_move, fire_out, tail_fn):
    nfix = KMAX - 1
    ixd = [None] * nfix
    for k in range(nfix):
        off = (wid + NW * k) * CH
        ixd[k] = pltpu.async_copy(idx_hbm.at[pl.ds(off, CH)], idx_refs[k],
                                  isem)
    for k in range(nfix):
        ixd[k].wait()

    md, sd = [None] * nfix, [None] * nfix
    for k in range(3):
        md[k] = fire_move(k, k % 3)
    for k in range(nfix):
        md[k].wait()
        sd[k] = fire_out(k, k % 3)
        if k + 3 < nfix:
            sd[k].wait()
            md[k + 3] = fire_move(k + 3, (k + 3) % 3)
    for k in range(max(0, nfix - 3), nfix):
        sd[k].wait()

    @pl.when(wid < _NLAST)
    def _():
        off = (wid + NW * nfix) * CH
        pltpu.sync_copy(idx_hbm.at[pl.ds(off, CH)], idx_refs[nfix])
        md4 = fire_move(nfix, 0)
        md4.wait()
        fire_out(nfix, 0).wait()

    @pl.when(wid == NW - 1)
    def _():
        pltpu.sync_copy(idx_hbm.at[pl.ds(NFULL * CH, TAIL)], it)
        tail_fn()


@functools.lru_cache(maxsize=1)
def _sc_sort_scatter_fn():
    @functools.partial(
        pl.kernel,
        mesh=_mesh(),
        out_type=jax.ShapeDtypeStruct((BP, D_EMB), jnp.float32),
        scratch_types=list(_RING_SCRATCH),
    )
    def _go(rep_hbm, pos_hbm, out_hbm, i0, i1, i2, i3, i4, it, b0, b1, b2,
            bt, isem, r0, r1, r2, s0, s1, s2, lit, st):
        wid = lax.axis_index("s") * 2 + lax.axis_index("c")
        idx_refs, bufs = (i0, i1, i2, i3, i4), (b0, b1, b2)
        rsems, ssems = (r0, r1, r2), (s0, s1, s2)

        def fire_move(k, r):
            off = (wid + NW * k) * CH
            return pltpu.async_copy(rep_hbm.at[pl.ds(off, CH), :], bufs[r],
                                    rsems[r])

        def fire_out(k, r):
            return pltpu.async_copy(bufs[r], out_hbm.at[idx_refs[k]],
                                    ssems[r])

        def tail_fn():
            off = NFULL * CH
            pltpu.async_copy(rep_hbm.at[pl.ds(off, TAIL), :], bt, lit).wait()
            pltpu.async_copy(bt, out_hbm.at[it], st).wait()

        _ring_pipeline(wid, pos_hbm, idx_refs, it, bufs, bt, isem, rsems,
                       ssems, lit, st, fire_move, fire_out, tail_fn)

    return _go


@functools.lru_cache(maxsize=1)
def _sc_lookup_fn():
    @functools.partial(
        pl.kernel,
        mesh=_mesh(),
        out_type=jax.ShapeDtypeStruct((B, D_EMB), jnp.float32),
        scratch_types=list(_RING_SCRATCH),
    )
    def _go(w_hbm, enc_hbm, out_hbm, i0, i1, i2, i3, i4, it, b0, b1, b2,
            bt, isem, r0, r1, r2, s0, s1, s2, lit, st):
        wid = lax.axis_index("s") * 2 + lax.axis_index("c")
        idx_refs, bufs = (i0, i1, i2, i3, i4), (b0, b1, b2)
        rsems, ssems = (r0, r1, r2), (s0, s1, s2)

        def fire_move(k, r):
            return pltpu.async_copy(w_hbm.at[idx_refs[k]], bufs[r], rsems[r])

        def fire_out(k, r):
            off = (wid + NW * k) * CH
            return pltpu.async_copy(bufs[r], out_hbm.at[pl.ds(off, CH), :],
                                    ssems[r])

        def tail_fn():
            off = NFULL * CH
            pltpu.async_copy(w_hbm.at[it], bt, lit).wait()
            pltpu.async_copy(bt, out_hbm.at[pl.ds(off, TAIL), :], st).wait()

        _ring_pipeline(wid, enc_hbm, idx_refs, it, bufs, bt, isem, rsems,
                       ssems, lit, st, fire_move, fire_out, tail_fn)

    return _go


def kernel(node_type, node_representation, W):
    rep = node_representation.astype(jnp.float32)
    w = W.astype(jnp.float32)
    nt = node_type.astype(jnp.int32)
    g = jnp.where(nt == 5, 0, jnp.where(nt == 6, 1, jnp.where(nt == 7, 2, 3)))

    a = jnp.where(g == 0, 1, 0) + jnp.where(g == 1, 1 << 16, 0)
    b = jnp.where(g == 2, 1, 0) + jnp.where(g == 3, 1 << 16, 0)
    ca = jnp.cumsum(a)
    cb = jnp.cumsum(b)
    m16 = jnp.int32(0xFFFF)
    totals = jnp.stack([ca[B - 1] & m16, ca[B - 1] >> 16,
                        cb[B - 1] & m16, cb[B - 1] >> 16])
    offsets = jnp.concatenate(
        [jnp.zeros((1,), jnp.int32), jnp.cumsum(totals)[:3]])
    rank = jnp.where(g == 0, ca & m16,
                     jnp.where(g == 1, ca >> 16,
                               jnp.where(g == 2, cb & m16, cb >> 16))) - 1
    pos = offsets[g] + rank
    co = jnp.cumsum(totals)[:3]
    gs = jnp.sum(
        (jnp.arange(B, dtype=jnp.int32)[:, None] >= co[None, :]),
        axis=1).astype(jnp.int32)

    wsq = jnp.sum(w ** 2, axis=1)

    rep_s = _sc_sort_scatter_fn()(rep, pos)

    pad_g = jnp.full((BP - B,), -1, jnp.int32)
    gs3 = jnp.concatenate([gs, pad_g]).reshape(NT, 1, TR)
    tstart = jnp.arange(NT, dtype=jnp.int32) * TR
    lo = gs[jnp.minimum(tstart, B - 1)]
    hi = gs[jnp.minimum(tstart + TR - 1, B - 1)]
    lohi = jnp.stack([lo, hi])

    w4 = (w + w).reshape(4, QK, D_EMB)
    wsq43 = wsq.reshape(4, 1, QK)
    enc_s, loss_sum = _encode(lohi, gs3, rep_s, w4, wsq43)
    enc = enc_s[pos]

    quantized = _sc_lookup_fn()(w, enc)
    vq_loss = loss_sum / jnp.float32(B * D_EMB)
    commitment_loss = jnp.float32(0.25) * vq_loss
    return (vq_loss, commitment_loss, enc, quantized)
